# baseline (speedup 1.0000x reference)
import functools

import jax
import jax.numpy as jnp
from jax import lax
from jax.experimental import pallas as pl
from jax.experimental.pallas import tpu as pltpu
from jax.experimental.pallas import tpu_sc as plsc

B, TOPK, E, D_MODEL, D_FF = 512, 2, 64, 1024, 64
KA = E * D_FF
EPB = 32
GRID = E // EPB
KBLK = EPB * D_FF
NW = 32
TPW = B // NW
XROWS = TOPK * D_FF + 2 * TOPK


def _sc_dispatch_body(x_hbm, a_hbm, x_v, a_v):
    wid = lax.axis_index("s") * 2 + lax.axis_index("c")
    base = wid * TPW
    pltpu.sync_copy(x_hbm.at[wid], x_v)

    w0 = x_v[128, :]
    w1 = x_v[129, :]
    idx0 = x_v[130, :].astype(jnp.int32)
    idx1 = x_v[131, :].astype(jnp.int32)
    eqf = jnp.where(idx0 == idx1, 1.0, 0.0)

    zero16 = jnp.zeros((16,), jnp.float32)

    def _zbody(j, carry):
        for i in range(16):
            a_v[pl.ds(j * 256 + i * 16, 16)] = zero16
        return carry

    lax.fori_loop(0, TPW * KA // 256, _zbody, 0)

    row_base = lax.iota(jnp.int32, 16) * KA
    for t in range(TOPK):
        idx_t = idx0 if t == 0 else idx1
        w_t = w0 if t == 0 else w1
        w_o = w1 if t == 0 else w0
        for c in range(D_FF):
            av = x_v[t * D_FF + c, :]
            ov = x_v[(1 - t) * D_FF + c, :]
            val = av * w_t + eqf * ov * w_o
            cols = row_base + idx_t * D_FF + c
            plsc.store_scatter(a_v, [cols], val)

    pltpu.sync_copy(a_v, a_hbm.at[pl.ds(base * KA, TPW * KA)])


_sc_dispatch = functools.partial(
    pl.kernel,
    mesh=plsc.VectorSubcoreMesh(core_axis_name="c", subcore_axis_name="s"),
    out_type=jax.ShapeDtypeStruct((B * KA,), jnp.float32),
    compiler_params=pltpu.CompilerParams(needs_layout_passes=False),
    scratch_types=[
        pltpu.VMEM((XROWS, TPW), jnp.float32),
        pltpu.VMEM((TPW * KA,), jnp.float32),
    ],
)(_sc_dispatch_body)


def _moe_body(a_ref, idx_ref, wgt_ref, w_ref, bias_ref, resid_ref, out_ref):
    g = pl.program_id(0)

    @pl.when(g == 0)
    def _init():
        wgt = wgt_ref[...]
        idx = idx_ref[...]
        eids = lax.broadcasted_iota(jnp.int32, (B, TOPK, E), 2)
        cb = jnp.sum(jnp.where(idx[:, :, None] == eids,
                               wgt[:, :, None], 0.0),
                     axis=1).astype(jnp.bfloat16)
        out_ref[...] = resid_ref[...] + jnp.dot(
            cb, bias_ref[...], preferred_element_type=jnp.float32)

    a_blk = a_ref[...].astype(jnp.bfloat16)
    out_ref[...] += jnp.dot(a_blk, w_ref[...],
                            preferred_element_type=jnp.float32)


def kernel(activated, expert_indices, expert_weights, mlp2_weight, mlp2_bias,
           residual_x):
    idx32 = expert_indices.astype(jnp.int32)
    x = jnp.concatenate(
        [activated.reshape(B, TOPK * D_FF).T,
         expert_weights.T,
         expert_indices.astype(jnp.float32).T], axis=0)
    x = x.reshape(XROWS, NW, TPW).transpose(1, 0, 2)
    a = _sc_dispatch(x).reshape(B, KA)
    w2 = jnp.swapaxes(mlp2_weight, 1, 2).reshape(E * D_FF, D_MODEL)
    w2_bf = w2.astype(jnp.bfloat16)
    bias_bf = mlp2_bias.astype(jnp.bfloat16)
    return pl.pallas_call(
        _moe_body,
        grid=(GRID,),
        in_specs=[
            pl.BlockSpec((B, KBLK), lambda g: (0, g)),
            pl.BlockSpec((B, TOPK), lambda g: (0, 0)),
            pl.BlockSpec((B, TOPK), lambda g: (0, 0)),
            pl.BlockSpec((KBLK, D_MODEL), lambda g: (g, 0)),
            pl.BlockSpec((E, D_MODEL), lambda g: (0, 0)),
            pl.BlockSpec((B, D_MODEL), lambda g: (0, 0)),
        ],
        out_specs=pl.BlockSpec((B, D_MODEL), lambda g: (0, 0)),
        out_shape=jax.ShapeDtypeStruct((B, D_MODEL), jnp.float32),
    )(a, idx32, expert_weights, w2_bf, bias_bf, residual_x)


# speedup vs baseline: 1.9687x; 1.9687x over previous
import jax
import jax.numpy as jnp
from jax import lax
from jax.experimental import pallas as pl
from jax.experimental.pallas import tpu as pltpu

B, TOPK, E, D_MODEL, D_FF = 512, 2, 64, 1024, 64
EPB = 32
GRID = E // EPB
KBLK = EPB * D_FF


def _moe_body(act_ref, idx_ref, wgt_ref, w_ref, bias_ref, resid_ref, out_ref,
              a0_ref, a1_ref, j2_ref):
    g = pl.program_id(0)

    @pl.when(g == 0)
    def _init():
        wgt = wgt_ref[...]
        a0 = (act_ref[:, 0:D_FF] * wgt[:, 0:1]).astype(jnp.bfloat16)
        a1 = (act_ref[:, D_FF:2 * D_FF] * wgt[:, 1:2]).astype(jnp.bfloat16)
        a0_ref[...] = jnp.tile(a0, (1, EPB))
        a1_ref[...] = jnp.tile(a1, (1, EPB))
        cols = lax.broadcasted_iota(jnp.int32, (B, KBLK), 1)
        j2_ref[...] = lax.shift_right_logical(cols, 6)

        idx = idx_ref[...]
        eids = lax.broadcasted_iota(jnp.int32, (B, TOPK, E), 2)
        cb = jnp.sum(jnp.where(idx[:, :, None] == eids,
                               wgt[:, :, None], 0.0),
                     axis=1).astype(jnp.bfloat16)
        out_ref[...] = resid_ref[...] + jnp.dot(
            cb, bias_ref[...], preferred_element_type=jnp.float32)

    e0 = g * EPB
    j2 = j2_ref[...]
    d0 = idx_ref[:, 0:1] - e0
    d1 = idx_ref[:, 1:2] - e0
    zero = jnp.zeros((), jnp.bfloat16)
    a_blk = (jnp.where(j2 == d0, a0_ref[...], zero)
             + jnp.where(j2 == d1, a1_ref[...], zero))
    out_ref[...] += jnp.dot(a_blk, w_ref[...],
                            preferred_element_type=jnp.float32)


def kernel(activated, expert_indices, expert_weights, mlp2_weight, mlp2_bias,
           residual_x):
    idx32 = expert_indices.astype(jnp.int32)
    act2 = activated.reshape(B, TOPK * D_FF)
    w2 = jnp.swapaxes(mlp2_weight, 1, 2).reshape(E * D_FF, D_MODEL)
    w2_bf = w2.astype(jnp.bfloat16)
    bias_bf = mlp2_bias.astype(jnp.bfloat16)
    return pl.pallas_call(
        _moe_body,
        grid=(GRID,),
        in_specs=[
            pl.BlockSpec((B, TOPK * D_FF), lambda g: (0, 0)),
            pl.BlockSpec((B, TOPK), lambda g: (0, 0)),
            pl.BlockSpec((B, TOPK), lambda g: (0, 0)),
            pl.BlockSpec((KBLK, D_MODEL), lambda g: (g, 0)),
            pl.BlockSpec((E, D_MODEL), lambda g: (0, 0)),
            pl.BlockSpec((B, D_MODEL), lambda g: (0, 0)),
        ],
        out_specs=pl.BlockSpec((B, D_MODEL), lambda g: (0, 0)),
        out_shape=jax.ShapeDtypeStruct((B, D_MODEL), jnp.float32),
        scratch_shapes=[
            pltpu.VMEM((B, KBLK), jnp.bfloat16),
            pltpu.VMEM((B, KBLK), jnp.bfloat16),
            pltpu.VMEM((B, KBLK), jnp.int32),
        ],
    )(act2, idx32, expert_weights, w2_bf, bias_bf, residual_x)
